# Initial kernel scaffold; baseline (speedup 1.0000x reference)
import jax
import jax.numpy as jnp
from jax.experimental import pallas as pl


def kernel(x, edge_index, W_in, b_in, phi_w1_0, phi_b1_0, phi_w2_0, phi_b2_0, phi_w1_1, phi_b1_1, phi_w2_1, phi_b2_1, lt_w_0, lt_b_0, lt_w_1, lt_b_1, W_out, b_out):
    raise NotImplementedError("write your pallas kernel here")


# speedup vs baseline: 7.1482x; 7.1482x over previous
import functools

import numpy as np
import jax
import jax.numpy as jnp
from jax import lax
from jax.experimental import pallas as pl
from jax.experimental.pallas import tpu as pltpu
from jax.experimental.pallas import tpu_sc as plsc

_N = 10000
_E = 320000
_D = 128
_B = 64
_K = 8
_T = 1.0

_NP = 10240
_NCORES = 2
_NSUB = 16
_NTILES = _NCORES * _NSUB
_RPT = _NP // _NSUB
_C = 128
_NCH = 79
_EPT = _NCH * _C
_EP = _EPT * _NTILES
_BLK = 2048


def _sc_mesh():
    return plsc.VectorSubcoreMesh(
        core_axis_name="c", subcore_axis_name="s",
        num_cores=_NCORES, num_subcores=_NSUB)


def _sc_step(src3, dst3, term):
    @functools.partial(
        pl.kernel,
        out_type=jax.ShapeDtypeStruct((_NCORES * _NP, _D), jnp.float32),
        mesh=_sc_mesh(),
        scratch_types=[
            pltpu.VMEM((_NCH, _C), jnp.int32),
            pltpu.VMEM((_NCH, _C), jnp.int32),
            pltpu.VMEM((_C, _D), jnp.float32),
            pltpu.VMEM_SHARED((_NP, _D), jnp.float32),
        ],
    )
    def k(src_h, dst_h, term_h, out_h, sidx, didx, rows, agg_sh):
        c = lax.axis_index("c")
        s = lax.axis_index("s")
        wid = c * _NSUB + s

        def zrow(r, carry):
            for j in range(_D // 16):
                rows[r, pl.ds(j * 16, 16)] = jnp.zeros((16,), jnp.float32)
            return carry
        lax.fori_loop(0, _C, zrow, 0)

        base = s * _RPT

        def zcp(j, carry):
            pltpu.sync_copy(rows, agg_sh.at[pl.ds(base + j * _C, _C)])
            return carry
        lax.fori_loop(0, _RPT // _C, zcp, 0)

        pltpu.sync_copy(src_h.at[wid], sidx)
        pltpu.sync_copy(dst_h.at[wid], didx)
        plsc.subcore_barrier()

        def step(i, carry):
            pltpu.sync_copy(term_h.at[sidx.at[i]], rows)
            pltpu.sync_copy(rows, agg_sh.at[didx.at[i]], add=True)
            return carry
        lax.fori_loop(0, _NCH, step, 0)
        plsc.subcore_barrier()

        out_base = c * _NP + base

        def wcp(j, carry):
            pltpu.sync_copy(agg_sh.at[pl.ds(base + j * _C, _C)], rows)
            pltpu.sync_copy(rows, out_h.at[pl.ds(out_base + j * _C, _C)])
            return carry
        lax.fori_loop(0, _RPT // _C, wcp, 0)

    return k(src3, dst3, term)


_SQRT1_2 = 0.7071067811865476


def _gelu(x):
    return 0.5 * x * (1.0 + lax.erf(x * _SQRT1_2))


def _full(shape):
    return pl.BlockSpec(shape, lambda i: (0,) * len(shape))


def _rows(d):
    return pl.BlockSpec((_BLK, d), lambda i: (i, 0))


def _tc_linear(xt, W, b):
    dout, din = W.shape

    def body(x_ref, w_ref, b_ref, o_ref):
        o_ref[...] = (
            jnp.dot(x_ref[...], w_ref[...].T, preferred_element_type=jnp.float32)
            + b_ref[...])

    return pl.pallas_call(
        body,
        grid=(_NP // _BLK,),
        in_specs=[_rows(din), _full((dout, din)), _full((1, dout))],
        out_specs=_rows(dout),
        out_shape=jax.ShapeDtypeStruct((_NP, dout), jnp.float32),
    )(xt, W, b)


def _tc_prologue(ht, w1, b1, w2, b2, wl, bl):
    def body(h_ref, w1_ref, b1_ref, w2_ref, b2_ref, wl_ref, bl_ref,
             H_ref, c_ref, s_ref):
        hb = h_ref[...]
        a = _gelu(jnp.dot(hb, w1_ref[...].T, preferred_element_type=jnp.float32)
                  + b1_ref[...])
        ang = (jnp.dot(a, w2_ref[...].T, preferred_element_type=jnp.float32)
               + b2_ref[...])
        cv = jnp.cos(ang)
        sv = jnp.sin(ang)
        X = hb[:, :_B]
        Y = hb[:, _B:]
        hr = jnp.concatenate([cv * X - sv * Y, sv * X + cv * Y], axis=1)
        H_ref[...] = (jnp.dot(hr, wl_ref[...].T,
                              preferred_element_type=jnp.float32) + bl_ref[...])
        c_ref[...] = cv
        s_ref[...] = sv

    return pl.pallas_call(
        body,
        grid=(_NP // _BLK,),
        in_specs=[_rows(_D), _full((_D, _D)), _full((1, _D)),
                  _full((_B, _D)), _full((1, _B)),
                  _full((_D, _D)), _full((1, _D))],
        out_specs=[_rows(_D), _rows(_B), _rows(_B)],
        out_shape=[jax.ShapeDtypeStruct((_NP, _D), jnp.float32),
                   jax.ShapeDtypeStruct((_NP, _B), jnp.float32),
                   jax.ShapeDtypeStruct((_NP, _B), jnp.float32)],
    )(ht, w1, b1, w2, b2, wl, bl)


def _tc_recip(deg2):
    nb = _NP // _BLK

    def body(d0_ref, d1_ref, o_ref):
        d = d0_ref[:, 0:1] + d1_ref[:, 0:1]
        o_ref[...] = 1.0 / jnp.maximum(d, 1.0)

    return pl.pallas_call(
        body,
        grid=(nb,),
        in_specs=[pl.BlockSpec((_BLK, _D), lambda i: (i, 0)),
                  pl.BlockSpec((_BLK, _D), lambda i: (i + nb, 0))],
        out_specs=_rows(1),
        out_shape=jax.ShapeDtypeStruct((_NP, 1), jnp.float32),
    )(deg2, deg2)


def _tc_update(term, aggs, rec, result, coef):
    def body(t_ref, a0_ref, a1_ref, r_ref, res_ref, tn_ref, rn_ref):
        agg = a0_ref[...] + a1_ref[...]
        tn = coef * (t_ref[...] - agg * r_ref[...])
        tn_ref[...] = tn
        rn_ref[...] = res_ref[...] + tn

    nb = _NP // _BLK
    return pl.pallas_call(
        body,
        grid=(nb,),
        in_specs=[_rows(_D),
                  pl.BlockSpec((_BLK, _D), lambda i: (i, 0)),
                  pl.BlockSpec((_BLK, _D), lambda i: (i + nb, 0)),
                  _rows(1), _rows(_D)],
        out_specs=[_rows(_D), _rows(_D)],
        out_shape=[jax.ShapeDtypeStruct((_NP, _D), jnp.float32),
                   jax.ShapeDtypeStruct((_NP, _D), jnp.float32)],
    )(term, aggs, aggs, rec, result)


def _tc_epilogue(result, cv, sv, ht):
    def body(r_ref, c_ref, s_ref, h_ref, o_ref):
        cb = c_ref[...]
        sb = s_ref[...]
        Xr = r_ref[:, :_B]
        Yr = r_ref[:, _B:]
        ho = jnp.concatenate([cb * Xr + sb * Yr, -sb * Xr + cb * Yr], axis=1)
        o_ref[...] = h_ref[...] + _gelu(ho)

    return pl.pallas_call(
        body,
        grid=(_NP // _BLK,),
        in_specs=[_rows(_D), _rows(_B), _rows(_B), _rows(_D)],
        out_specs=_rows(_D),
        out_shape=jax.ShapeDtypeStruct((_NP, _D), jnp.float32),
    )(result, cv, sv, ht)


_PERM = np.concatenate([np.arange(0, _D, 2), np.arange(1, _D, 2)])


def kernel(x, edge_index, W_in, b_in,
           phi_w1_0, phi_b1_0, phi_w2_0, phi_b2_0,
           phi_w1_1, phi_b1_1, phi_w2_1, phi_b2_1,
           lt_w_0, lt_b_0, lt_w_1, lt_b_1,
           W_out, b_out):
    p = _PERM
    W_in2 = W_in[p, :]
    b_in2 = b_in[p][None]
    phis = [(phi_w1_0[:, p], phi_b1_0[None], phi_w2_0, phi_b2_0[None]),
            (phi_w1_1[:, p], phi_b1_1[None], phi_w2_1, phi_b2_1[None])]
    lts = [(lt_w_0[p][:, p], lt_b_0[p][None]),
           (lt_w_1[p][:, p], lt_b_1[p][None])]
    W_out2 = W_out[:, p]

    xp = jnp.pad(x, ((0, _NP - _N), (0, 0)))
    src = edge_index[0]
    dst = edge_index[1]
    padidx = _N + (jnp.arange(_EP - _E, dtype=jnp.int32) % (_NP - _N))
    src3 = jnp.concatenate([src, padidx]).reshape(_NTILES, _NCH, _C)
    dst3 = jnp.concatenate([dst, padidx]).reshape(_NTILES, _NCH, _C)

    deg2 = _sc_step(src3, dst3, jnp.ones((_NP, _D), jnp.float32))
    rec = _tc_recip(deg2)

    ht = _tc_linear(xp, W_in2, b_in2)
    for l in range(2):
        w1, b1, w2, b2 = phis[l]
        wl, bl = lts[l]
        Ht, cv, sv = _tc_prologue(ht, w1, b1, w2, b2, wl, bl)
        term = Ht
        result = Ht
        for k in range(1, _K + 1):
            aggs = _sc_step(src3, dst3, term)
            term, result = _tc_update(term, aggs, rec, result, -_T / k)
        ht = _tc_epilogue(result, cv, sv, ht)

    out = _tc_linear(ht, W_out2, b_out[None])
    return out[:_N]
